# baseline (speedup 1.0000x reference)
import functools

import jax
import jax.numpy as jnp
from jax import lax
from jax.experimental import pallas as pl
from jax.experimental.pallas import tpu as pltpu
from jax.experimental.pallas import tpu_sc as plsc

B, S, D = 16384, 6, 512
TCB = 5120
SCB = B - TCB
NC, NS = 2, 16
NW = NC * NS
BPW = SCB // NW
CB = 44
NCHB = BPW // CB
BB = 512


@functools.partial(
    pl.kernel,
    out_type=jax.ShapeDtypeStruct((SCB, S, D), jnp.float32),
    mesh=plsc.VectorSubcoreMesh(core_axis_name="c", subcore_axis_name="s"),
    scratch_types=[
        pltpu.VMEM((16,), jnp.int32),
        pltpu.VMEM((CB, 1, D), jnp.float32),
        pltpu.VMEM((CB, 1, D), jnp.float32),
        pltpu.SemaphoreType.DMA,
        pltpu.SemaphoreType.DMA,
        pltpu.SemaphoreType.DMA,
        pltpu.SemaphoreType.DMA,
    ],
    compiler_params=pltpu.CompilerParams(
        use_tc_tiling_on_sc=True, needs_layout_passes=False),
)
def _sc_shuffle(x_hbm, tab_hbm, out_hbm, tab_v, buf0, buf1, g0, g1, w0, w1):
    wid = lax.axis_index("s") * NC + lax.axis_index("c")
    b0 = wid * BPW

    pltpu.sync_copy(tab_hbm, tab_v)
    tab_vec = tab_v[...]
    iota = lax.broadcasted_iota(jnp.int32, (16,), 0)
    pis = [jnp.max(jnp.where(iota == i, tab_vec, 0)) for i in range(S)]

    tasks = [(i, c) for i in range(S) for c in range(NCHB)]
    ntask = len(tasks)
    buf = (buf0, buf1)
    gsem = (g0, g1)
    wsem = (w0, w1)

    def gather(t):
        i, c = tasks[t]
        return pltpu.async_copy(
            x_hbm.at[pl.ds(b0 + c * CB, CB), pl.ds(pis[i], 1), :],
            buf[t % 2], gsem[t % 2])

    def write(t):
        i, c = tasks[t]
        return pltpu.async_copy(
            buf[t % 2],
            out_hbm.at[pl.ds(b0 + c * CB, CB), pl.ds(i, 1), :],
            wsem[t % 2])

    gh = [None, None]
    wh = [None, None]
    gh[0] = gather(0)
    for t in range(ntask):
        b = t % 2
        gh[b].wait()
        wh[b] = write(t)
        if t + 1 < ntask:
            nb = (t + 1) % 2
            if wh[nb] is not None:
                wh[nb].wait()
            gh[nb] = gather(t + 1)
    wh[0].wait()
    wh[1].wait()


def _tc_body(idx_ref, x_ref, o_ref):
    idxv = jnp.stack([idx_ref[i] for i in range(S)])
    idx3 = jnp.broadcast_to(idxv[None, :, None], (BB, S, D))
    o_ref[...] = jnp.take_along_axis(x_ref[...], idx3, axis=1)


def _tc_shuffle(x, idx):
    return pl.pallas_call(
        _tc_body,
        grid_spec=pltpu.PrefetchScalarGridSpec(
            num_scalar_prefetch=1,
            grid=(TCB // BB,),
            in_specs=[pl.BlockSpec((BB, S, D), lambda b, idx_ref: (b, 0, 0))],
            out_specs=pl.BlockSpec((BB, S, D), lambda b, idx_ref: (b, 0, 0)),
        ),
        out_shape=jax.ShapeDtypeStruct((B, S, D), jnp.float32),
    )(idx, x)


def kernel(x, index):
    idx32 = index.astype(jnp.int32)
    tab16 = jnp.zeros((16,), jnp.int32).at[:S].set(idx32)
    sc_out = _sc_shuffle(lax.slice(x, (TCB, 0, 0), (B, S, D)), tab16)
    tc_out = _tc_shuffle(x, idx32)
    return lax.dynamic_update_slice(tc_out, sc_out, (TCB, 0, 0))


# speedup vs baseline: 1.4784x; 1.4784x over previous
import functools

import jax
import jax.numpy as jnp
from jax import lax
from jax.experimental import pallas as pl
from jax.experimental.pallas import tpu as pltpu
from jax.experimental.pallas import tpu_sc as plsc

B, S, D = 16384, 6, 512
NC, NS = 2, 16
NW = NC * NS
BPW = B // NW
CB = 64
NCHB = BPW // CB


@functools.partial(
    pl.kernel,
    out_type=jax.ShapeDtypeStruct((B, S, D), jnp.float32),
    mesh=plsc.VectorSubcoreMesh(core_axis_name="c", subcore_axis_name="s"),
    scratch_types=[
        pltpu.VMEM((16,), jnp.int32),
        pltpu.VMEM((CB, 1, D), jnp.float32),
        pltpu.VMEM((CB, 1, D), jnp.float32),
        pltpu.SemaphoreType.DMA,
        pltpu.SemaphoreType.DMA,
        pltpu.SemaphoreType.DMA,
        pltpu.SemaphoreType.DMA,
    ],
    compiler_params=pltpu.CompilerParams(
        use_tc_tiling_on_sc=True, needs_layout_passes=False),
)
def _shuffle_slabs(x_hbm, tab_hbm, out_hbm, tab_v, buf0, buf1,
                   g0, g1, w0, w1):
    wid = lax.axis_index("s") * NC + lax.axis_index("c")
    b0 = wid * BPW

    pltpu.sync_copy(tab_hbm, tab_v)
    tab_vec = tab_v[...]
    iota = lax.broadcasted_iota(jnp.int32, (16,), 0)
    pis = [jnp.max(jnp.where(iota == i, tab_vec, 0)) for i in range(S)]

    tasks = [(i, c) for c in range(NCHB) for i in range(S)]
    ntask = len(tasks)
    buf = (buf0, buf1)
    gsem = (g0, g1)
    wsem = (w0, w1)

    def gather(t):
        i, c = tasks[t]
        return pltpu.async_copy(
            x_hbm.at[pl.ds(b0 + c * CB, CB), pl.ds(pis[i], 1), :],
            buf[t % 2], gsem[t % 2])

    def write(t):
        i, c = tasks[t]
        return pltpu.async_copy(
            buf[t % 2],
            out_hbm.at[pl.ds(b0 + c * CB, CB), pl.ds(i, 1), :],
            wsem[t % 2])

    gh = [None, None]
    wh = [None, None]
    gh[0] = gather(0)
    for t in range(ntask):
        b = t % 2
        gh[b].wait()
        wh[b] = write(t)
        if t + 1 < ntask:
            nb = (t + 1) % 2
            if wh[nb] is not None:
                wh[nb].wait()
            gh[nb] = gather(t + 1)
    wh[0].wait()
    wh[1].wait()


def kernel(x, index):
    tab16 = jnp.zeros((16,), jnp.int32).at[:S].set(index.astype(jnp.int32))
    return _shuffle_slabs(x, tab16)


# speedup vs baseline: 1.4895x; 1.0075x over previous
import functools

import jax
import jax.numpy as jnp
from jax import lax
from jax.experimental import pallas as pl
from jax.experimental.pallas import tpu as pltpu
from jax.experimental.pallas import tpu_sc as plsc

B, S, D = 16384, 6, 512
NC, NS = 2, 16
NW = NC * NS
BPW = B // NW
CB = 64
NCHB = BPW // CB


@functools.partial(
    pl.kernel,
    out_type=jax.ShapeDtypeStruct((B, S, D), jnp.float32),
    mesh=plsc.VectorSubcoreMesh(core_axis_name="c", subcore_axis_name="s"),
    scratch_types=[
        pltpu.VMEM((16,), jnp.int32),
        pltpu.VMEM((CB, 1, D), jnp.float32),
        pltpu.VMEM((CB, 1, D), jnp.float32),
        pltpu.SemaphoreType.DMA,
        pltpu.SemaphoreType.DMA,
        pltpu.SemaphoreType.DMA,
        pltpu.SemaphoreType.DMA,
    ],
    compiler_params=pltpu.CompilerParams(
        use_tc_tiling_on_sc=True, needs_layout_passes=False),
)
def _shuffle_slabs(x_hbm, tab_hbm, out_hbm, tab_v, buf0, buf1,
                   g0, g1, w0, w1):
    wid = lax.axis_index("s") * NC + lax.axis_index("c")
    b0 = wid * BPW

    pltpu.sync_copy(tab_hbm, tab_v)
    tab_vec = tab_v[...]
    iota = lax.broadcasted_iota(jnp.int32, (16,), 0)
    pis = [jnp.max(jnp.where(iota == i, tab_vec, 0)) for i in range(S)]

    tasks = [(i, c) for i in range(S) for c in range(NCHB)]
    ntask = len(tasks)
    buf = (buf0, buf1)
    gsem = (g0, g1)
    wsem = (w0, w1)

    def gather(t):
        i, c = tasks[t]
        return pltpu.async_copy(
            x_hbm.at[pl.ds(b0 + c * CB, CB), pl.ds(pis[i], 1), :],
            buf[t % 2], gsem[t % 2])

    def write(t):
        i, c = tasks[t]
        return pltpu.async_copy(
            buf[t % 2],
            out_hbm.at[pl.ds(b0 + c * CB, CB), pl.ds(i, 1), :],
            wsem[t % 2])

    gh = [None, None]
    wh = [None, None]
    gh[0] = gather(0)
    for t in range(ntask):
        b = t % 2
        gh[b].wait()
        wh[b] = write(t)
        if t + 1 < ntask:
            nb = (t + 1) % 2
            if wh[nb] is not None:
                wh[nb].wait()
            gh[nb] = gather(t + 1)
    wh[0].wait()
    wh[1].wait()


def kernel(x, index):
    tab16 = jnp.zeros((16,), jnp.int32).at[:S].set(index.astype(jnp.int32))
    return _shuffle_slabs(x, tab16)
